# baseline (speedup 1.0000x reference)
import jax
import jax.numpy as jnp
from jax.experimental import pallas as pl

TOP_K = 2
N_ROUTED = 8
SCALING = 2.5


def kernel(hidden_states, W_router, correction_bias, w_gate, w_up, w_down):
    xb = hidden_states.astype(jnp.bfloat16)
    logits = jax.lax.dot_general(
        xb, W_router.astype(jnp.bfloat16),
        (((1,), (1,)), ((), ())), preferred_element_type=jnp.float32)
    scores = jax.nn.softmax(logits, axis=-1)
    scores_for_choice = scores + correction_bias[None, :]
    topk_w, topk_ids = jax.lax.top_k(scores_for_choice, TOP_K)
    topk_w = topk_w.astype(jnp.float32) * SCALING
    zero_mask = topk_ids >= N_ROUTED
    zero_w = jnp.where(zero_mask, topk_w, 0.0)
    zero_expert_output = hidden_states * jnp.sum(zero_w, axis=-1, keepdims=True)
    topk_ids = jnp.where(zero_mask, 0, topk_ids)
    topk_w = jnp.where(zero_mask, 0.0, topk_w)
    n_tok = hidden_states.shape[0]
    combine = jnp.zeros((n_tok, N_ROUTED), jnp.float32).at[
        jnp.arange(n_tok)[:, None], topk_ids
    ].add(topk_w)
    gate = jnp.einsum('td,efd->tef', xb, w_gate.astype(jnp.bfloat16),
                      preferred_element_type=jnp.float32)
    up = jnp.einsum('td,efd->tef', xb, w_up.astype(jnp.bfloat16),
                    preferred_element_type=jnp.float32)
    act = jax.nn.silu(gate) * up
    act = act * combine[:, :, None]
    routed = jnp.einsum('tef,edf->td', act.astype(jnp.bfloat16),
                        w_down.astype(jnp.bfloat16),
                        preferred_element_type=jnp.float32)
    return routed + zero_expert_output


# speedup vs baseline: 1.1258x; 1.1258x over previous
import functools

import jax
import jax.numpy as jnp
from jax.experimental import pallas as pl
from jax.experimental.pallas import tpu as pltpu

T, D, DFF = 2048, 2048, 1024
NE = 8
NTOT = 16
SCALING = 2.5

_RT = 512
_BT = 512


def _router_body(x_ref, wr_ref, b_ref, comb_ref, zout_ref):
    x = x_ref[...]
    logits = jax.lax.dot_general(
        x.astype(jnp.bfloat16), wr_ref[...].astype(jnp.bfloat16),
        (((1,), (1,)), ((), ())), preferred_element_type=jnp.float32)
    m = jnp.max(logits, axis=1, keepdims=True)
    ex = jnp.exp(logits - m)
    p = ex / jnp.sum(ex, axis=1, keepdims=True)
    s = p + b_ref[...]
    iota = jax.lax.broadcasted_iota(jnp.int32, s.shape, 1)
    m1 = jnp.max(s, axis=1, keepdims=True)
    i1 = jnp.min(jnp.where(s == m1, iota, NTOT), axis=1, keepdims=True)
    s2 = jnp.where(iota == i1, -jnp.inf, s)
    m2 = jnp.max(s2, axis=1, keepdims=True)
    i2 = jnp.min(jnp.where(s2 == m2, iota, NTOT), axis=1, keepdims=True)
    w1 = m1 * SCALING
    w2 = m2 * SCALING
    zsum = (jnp.where(i1 >= NE, w1, 0.0) + jnp.where(i2 >= NE, w2, 0.0))
    iota8 = jax.lax.broadcasted_iota(jnp.int32, (x.shape[0], NE), 1)
    comb_ref[...] = (jnp.where(iota8 == i1, w1, 0.0)
                     + jnp.where(iota8 == i2, w2, 0.0))
    zout_ref[...] = x * zsum


def _dense_body(xb_ref, comb_ref, zout_ref, wg_ref, wu_ref, wd_ref, out_ref):
    e = pl.program_id(1)

    @pl.when(e == 0)
    def _():
        out_ref[...] = zout_ref[...]

    xb = xb_ref[...]
    gate = jax.lax.dot_general(xb, wg_ref[0], (((1,), (1,)), ((), ())),
                               preferred_element_type=jnp.float32)
    up = jax.lax.dot_general(xb, wu_ref[0], (((1,), (1,)), ((), ())),
                             preferred_element_type=jnp.float32)
    act = gate * (1.0 / (1.0 + jnp.exp(-gate))) * up
    iota8 = jax.lax.broadcasted_iota(jnp.int32, comb_ref.shape, 1)
    ce = jnp.sum(jnp.where(iota8 == e, comb_ref[...], 0.0), axis=1,
                 keepdims=True)
    act = act * ce
    out_ref[...] += jax.lax.dot_general(
        act.astype(jnp.bfloat16), wd_ref[0], (((1,), (1,)), ((), ())),
        preferred_element_type=jnp.float32)


def kernel(hidden_states, W_router, correction_bias, w_gate, w_up, w_down):
    comb, zout = pl.pallas_call(
        _router_body,
        grid=(T // _RT,),
        in_specs=[
            pl.BlockSpec((_RT, D), lambda t: (t, 0)),
            pl.BlockSpec((NTOT, D), lambda t: (0, 0)),
            pl.BlockSpec((1, NTOT), lambda t: (0, 0)),
        ],
        out_specs=[
            pl.BlockSpec((_RT, NE), lambda t: (t, 0)),
            pl.BlockSpec((_RT, D), lambda t: (t, 0)),
        ],
        out_shape=[
            jax.ShapeDtypeStruct((T, NE), jnp.float32),
            jax.ShapeDtypeStruct((T, D), jnp.float32),
        ],
    )(hidden_states, W_router, correction_bias.reshape(1, NTOT))

    xb = hidden_states.astype(jnp.bfloat16)
    wgb = w_gate.astype(jnp.bfloat16)
    wub = w_up.astype(jnp.bfloat16)
    wdb = w_down.astype(jnp.bfloat16)

    out = pl.pallas_call(
        _dense_body,
        grid=(T // _BT, NE),
        in_specs=[
            pl.BlockSpec((_BT, D), lambda t, e: (t, 0)),
            pl.BlockSpec((_BT, NE), lambda t, e: (t, 0)),
            pl.BlockSpec((_BT, D), lambda t, e: (t, 0)),
            pl.BlockSpec((1, DFF, D), lambda t, e: (e, 0, 0)),
            pl.BlockSpec((1, DFF, D), lambda t, e: (e, 0, 0)),
            pl.BlockSpec((1, D, DFF), lambda t, e: (e, 0, 0)),
        ],
        out_specs=pl.BlockSpec((_BT, D), lambda t, e: (t, 0)),
        out_shape=jax.ShapeDtypeStruct((T, D), jnp.float32),
    )(xb, comb, zout, wgb, wub, wdb)
    return out
